# baseline (speedup 1.0000x reference)
import functools

import jax
import jax.numpy as jnp
from jax import lax
from jax.experimental import pallas as pl
from jax.experimental.pallas import tpu as pltpu
from jax.experimental.pallas import tpu_sc as plsc

_VOCAB = 1000
_N = 51200
_NC, _NS, _LANES = 2, 16, 16
_NW = _NC * _NS
_RW = _N // _NW
_CH = 16
_NCH = _RW // _CH
_CW = _CH * _VOCAB // 128
_OR = _N * _VOCAB // 128


def _lse_body(tab_ref, out_ref):
    x = tab_ref[...]
    m = jnp.max(x, axis=1)
    s = jnp.sum(jnp.exp(x - m[:, None]), axis=1)
    out_ref[...] = jnp.log(s) + m


def _sum_body(p_ref, o_ref):
    o_ref[...] = (jnp.sum(p_ref[...]) / _N).reshape(1, 1)


def _sc_body(tab, idxf, tgtf, lse, out, part,
             idx_l, tgt_l, lse_l, accv,
             bufa, bufb, gsa, gsb, ssa, ssb):
    wid = lax.axis_index("s") * _NC + lax.axis_index("c")
    base = wid * _RW

    pltpu.sync_copy(idxf.at[pl.ds(base, _RW)], idx_l)
    pltpu.sync_copy(tgtf.at[pl.ds(base, _RW)], tgt_l)
    pltpu.sync_copy(lse, lse_l)

    iota = lax.iota(jnp.int32, _LANES)
    bufs, gs, ss = (bufa, bufb), (gsa, gsb), (ssa, ssb)

    def gdesc(c, p):
        return pltpu.make_async_copy(
            tab.at[idx_l.at[pl.ds(c * _CH, _CH)]], bufs[p], gs[p])

    def sstart(c, p):
        for r in range(_CH):
            pltpu.make_async_copy(
                bufs[p].at[r],
                out.at[pl.ds((base + c * _CH + r) * _VOCAB, _VOCAB)],
                ss[p]).start()

    def swait(c, p):
        for r in range(_CH):
            pltpu.make_async_copy(
                bufs[p].at[r],
                out.at[pl.ds((base + c * _CH + r) * _VOCAB, _VOCAB)],
                ss[p]).wait()

    def loss16(c, p, acc):
        i16 = idx_l[pl.ds(c * _CH, _CH)]
        t16 = tgt_l[pl.ds(c * _CH, _CH)]
        l16 = plsc.load_gather(lse_l, [i16])
        v16 = plsc.load_gather(bufs[p], [iota, t16])
        return acc + (l16 - v16)

    gdesc(0, 0).start()
    gdesc(0, 0).wait()
    sstart(0, 0)
    acc = loss16(0, 0, jnp.zeros((_LANES,), jnp.float32))
    gdesc(1, 1).start()

    def ring(o, acc):
        c1 = 2 * o + 1
        gdesc(c1, 1).wait()
        sstart(c1, 1)
        acc = loss16(c1, 1, acc)
        swait(c1 - 1, 0)
        gdesc(c1 + 1, 0).start()
        c2 = 2 * o + 2
        gdesc(c2, 0).wait()
        sstart(c2, 0)
        acc = loss16(c2, 0, acc)
        swait(c2 - 1, 1)
        gdesc(c2 + 1, 1).start()
        return acc
    acc = lax.fori_loop(0, (_NCH - 2) // 2, ring, acc)

    cl = _NCH - 1
    gdesc(cl, 1).wait()
    sstart(cl, 1)
    acc = loss16(cl, 1, acc)
    swait(cl - 1, 0)
    swait(cl, 1)

    accv[...] = acc
    pltpu.sync_copy(accv, part.at[pl.ds(wid * _LANES, _LANES)])


_sc_gather_loss = functools.partial(
    pl.kernel,
    out_type=(jax.ShapeDtypeStruct((_N * _VOCAB,), jnp.float32),
              jax.ShapeDtypeStruct((_NW * _LANES,), jnp.float32)),
    mesh=plsc.VectorSubcoreMesh(core_axis_name="c", subcore_axis_name="s",
                                num_cores=_NC, num_subcores=_NS),
    compiler_params=pltpu.CompilerParams(needs_layout_passes=False,
                                         use_tc_tiling_on_sc=False),
    scratch_types=[
        pltpu.VMEM((_RW,), jnp.int32),
        pltpu.VMEM((_RW,), jnp.int32),
        pltpu.VMEM((_VOCAB,), jnp.float32),
        pltpu.VMEM((_LANES,), jnp.float32),
        pltpu.VMEM((_CH, _VOCAB), jnp.float32),
        pltpu.VMEM((_CH, _VOCAB), jnp.float32),
        pltpu.SemaphoreType.DMA,
        pltpu.SemaphoreType.DMA,
        pltpu.SemaphoreType.DMA,
        pltpu.SemaphoreType.DMA,
    ],
)(_sc_body)


def kernel(idx, targets, table):
    idxf = idx.reshape(_N)
    tgtf = targets.reshape(_N)
    lse = pl.pallas_call(
        _lse_body,
        out_shape=jax.ShapeDtypeStruct((_VOCAB,), jnp.float32),
    )(table)
    blocks, part = _sc_gather_loss(table, idxf, tgtf, lse)
    loss = pl.pallas_call(
        _sum_body,
        out_shape=jax.ShapeDtypeStruct((1, 1), jnp.float32),
    )(part)
    return blocks.reshape(_N, _VOCAB), loss[0, 0]


# speedup vs baseline: 1.0834x; 1.0834x over previous
import functools

import jax
import jax.numpy as jnp
from jax import lax
from jax.experimental import pallas as pl
from jax.experimental.pallas import tpu as pltpu
from jax.experimental.pallas import tpu_sc as plsc

_VOCAB = 1000
_N = 51200
_NC, _NS, _LANES = 2, 16, 16
_NW = _NC * _NS
_RW = _N // _NW
_CH = 40
_NCH = _RW // _CH
_G = _RW // _LANES
_IDX_CH = 128


def _lse_body(tab_ref, out_ref):
    x = tab_ref[...]
    m = jnp.max(x, axis=1)
    s = jnp.sum(jnp.exp(x - m[:, None]), axis=1)
    out_ref[...] = jnp.log(s) + m


def _sum_body(p_ref, o_ref):
    o_ref[...] = (jnp.sum(p_ref[...]) / _N).reshape(1, 1)


def _sc_body(tab, idxf, tgtf, lse, out, part,
             idx_l, tgt_l, lse_l, accv,
             bufa, bufb, gsa, gsb, ssa, ssb):
    wid = lax.axis_index("s") * _NC + lax.axis_index("c")
    base = wid * _RW

    pltpu.sync_copy(idxf.at[pl.ds(base, _RW)], idx_l)
    pltpu.sync_copy(tgtf.at[pl.ds(base, _RW)], tgt_l)
    pltpu.sync_copy(lse, lse_l)

    iota = lax.iota(jnp.int32, _LANES)

    bufs, gs, ss = (bufa, bufb), (gsa, gsb), (ssa, ssb)

    def gdesc(c):
        p = c % 2
        return pltpu.make_async_copy(
            tab.at[idx_l.at[pl.ds(c * _CH, _CH)]], bufs[p], gs[p])

    def sdesc(c):
        p = c % 2
        return pltpu.make_async_copy(
            bufs[p], out.at[pl.ds(base + c * _CH, _CH)], ss[p])

    acc = jnp.zeros((_LANES,), jnp.float32)
    gdesc(0).start()
    for c in range(_NCH):
        if c + 1 < _NCH:
            if c >= 1:
                sdesc(c - 1).wait()
            gdesc(c + 1).start()
        gdesc(c).wait()
        sdesc(c).start()
        for h in range(_CH // _LANES + (1 if _CH % _LANES else 0)):
            nvalid = min(_LANES, _CH - h * _LANES)
            rid = h * _LANES + iota
            gid = c * _CH + rid
            if nvalid < _LANES:
                rid = jnp.minimum(rid, _CH - 1)
                gid = jnp.minimum(gid, _RW - 1)
            i16 = plsc.load_gather(idx_l, [gid])
            t16 = plsc.load_gather(tgt_l, [gid])
            l16 = plsc.load_gather(lse_l, [i16])
            v16 = plsc.load_gather(bufs[c % 2], [rid, t16])
            term = l16 - v16
            if nvalid < _LANES:
                term = jnp.where(iota < nvalid, term, 0.0)
            acc = acc + term
    sdesc(_NCH - 2).wait()
    sdesc(_NCH - 1).wait()

    accv[...] = acc
    pltpu.sync_copy(accv, part.at[pl.ds(wid * _LANES, _LANES)])


_sc_gather_loss = functools.partial(
    pl.kernel,
    out_type=(jax.ShapeDtypeStruct((_N, _VOCAB), jnp.float32),
              jax.ShapeDtypeStruct((_NW * _LANES,), jnp.float32)),
    mesh=plsc.VectorSubcoreMesh(core_axis_name="c", subcore_axis_name="s",
                                num_cores=_NC, num_subcores=_NS),
    compiler_params=pltpu.CompilerParams(needs_layout_passes=False,
                                         use_tc_tiling_on_sc=False),
    scratch_types=[
        pltpu.VMEM((_RW,), jnp.int32),
        pltpu.VMEM((_RW,), jnp.int32),
        pltpu.VMEM((_VOCAB,), jnp.float32),
        pltpu.VMEM((_LANES,), jnp.float32),
        pltpu.VMEM((_CH, _VOCAB), jnp.float32),
        pltpu.VMEM((_CH, _VOCAB), jnp.float32),
        pltpu.SemaphoreType.DMA,
        pltpu.SemaphoreType.DMA,
        pltpu.SemaphoreType.DMA,
        pltpu.SemaphoreType.DMA,
    ],
)(_sc_body)


def kernel(idx, targets, table):
    idxf = idx.reshape(_N)
    tgtf = targets.reshape(_N)
    lse = pl.pallas_call(
        _lse_body,
        out_shape=jax.ShapeDtypeStruct((_VOCAB,), jnp.float32),
    )(table)
    logits2d, part = _sc_gather_loss(table, idxf, tgtf, lse)
    loss = pl.pallas_call(
        _sum_body,
        out_shape=jax.ShapeDtypeStruct((1, 1), jnp.float32),
    )(part)
    return logits2d, loss[0, 0]


# speedup vs baseline: 1.0868x; 1.0031x over previous
import functools

import jax
import jax.numpy as jnp
from jax import lax
from jax.experimental import pallas as pl
from jax.experimental.pallas import tpu as pltpu
from jax.experimental.pallas import tpu_sc as plsc

_VOCAB = 1000
_N = 51200
_NC, _NS, _LANES = 2, 16, 16
_NW = _NC * _NS
_RW = _N // _NW
_CH = 32
_NCH = _RW // _CH
_NB = 3


def _lse_body(tab_ref, out_ref):
    x = tab_ref[...]
    m = jnp.max(x, axis=1)
    s = jnp.sum(jnp.exp(x - m[:, None]), axis=1)
    out_ref[...] = jnp.log(s) + m


def _sum_body(p_ref, o_ref):
    o_ref[...] = (jnp.sum(p_ref[...]) / _N).reshape(1, 1)


def _sc_body(tab, idxf, tgtf, lse, out, part,
             idx_l, tgt_l, lse_l, accv,
             bufa, bufb, bufc, gsa, gsb, gsc, ssa, ssb, ssc):
    wid = lax.axis_index("s") * _NC + lax.axis_index("c")
    base = wid * _RW

    pltpu.sync_copy(idxf.at[pl.ds(base, _RW)], idx_l)
    pltpu.sync_copy(tgtf.at[pl.ds(base, _RW)], tgt_l)
    pltpu.sync_copy(lse, lse_l)

    bufs, gs, ss = (bufa, bufb, bufc), (gsa, gsb, gsc), (ssa, ssb, ssc)

    def gdesc(c, p):
        return pltpu.make_async_copy(
            tab.at[idx_l.at[pl.ds(c * _CH, _CH)]], bufs[p], gs[p])

    def sdesc(c, p):
        return pltpu.make_async_copy(
            bufs[p], out.at[pl.ds(base + c * _CH, _CH)], ss[p])

    def loss32(c, p, acc):
        iota = lax.iota(jnp.int32, _LANES)
        for h in range(_CH // _LANES):
            s = pl.ds(c * _CH + h * _LANES, _LANES)
            i16 = idx_l[s]
            l16 = plsc.load_gather(lse_l, [i16])
            v16 = plsc.load_gather(bufs[p], [h * _LANES + iota, tgt_l[s]])
            acc = acc + (l16 - v16)
        return acc

    acc = jnp.zeros((_LANES,), jnp.float32)
    gdesc(0, 0).start()
    gdesc(1, 1).start()
    gdesc(0, 0).wait()
    sdesc(0, 0).start()
    acc = loss32(0, 0, acc)
    gdesc(2, 2).start()
    gdesc(1, 1).wait()
    sdesc(1, 1).start()
    acc = loss32(1, 1, acc)
    sdesc(0, 0).wait()
    gdesc(3, 0).start()

    def ring(o, acc):
        for j, p in ((2, 2), (3, 0), (4, 1)):
            c = 3 * o + j
            gdesc(c, p).wait()
            sdesc(c, p).start()
            acc = loss32(c, p, acc)
            sdesc(c - 1, (p - 1) % _NB).wait()
            gdesc(c + 2, (p + 2) % _NB).start()
        return acc
    acc = lax.fori_loop(0, (_NCH - 5) // 3, ring, acc)

    for c in range(_NCH - 3, _NCH):
        p = c % _NB
        gdesc(c, p).wait()
        sdesc(c, p).start()
        acc = loss32(c, p, acc)
        sdesc(c - 1, (p - 1) % _NB).wait()
        if c + 2 < _NCH:
            gdesc(c + 2, (c + 2) % _NB).start()
    sdesc(_NCH - 1, (_NCH - 1) % _NB).wait()

    accv[...] = acc
    pltpu.sync_copy(accv, part.at[pl.ds(wid * _LANES, _LANES)])


_sc_gather_loss = functools.partial(
    pl.kernel,
    out_type=(jax.ShapeDtypeStruct((_N, _VOCAB), jnp.float32),
              jax.ShapeDtypeStruct((_NW * _LANES,), jnp.float32)),
    mesh=plsc.VectorSubcoreMesh(core_axis_name="c", subcore_axis_name="s",
                                num_cores=_NC, num_subcores=_NS),
    compiler_params=pltpu.CompilerParams(needs_layout_passes=False,
                                         use_tc_tiling_on_sc=False),
    scratch_types=[
        pltpu.VMEM((_RW,), jnp.int32),
        pltpu.VMEM((_RW,), jnp.int32),
        pltpu.VMEM((_VOCAB,), jnp.float32),
        pltpu.VMEM((_LANES,), jnp.float32),
        pltpu.VMEM((_CH, _VOCAB), jnp.float32),
        pltpu.VMEM((_CH, _VOCAB), jnp.float32),
        pltpu.VMEM((_CH, _VOCAB), jnp.float32),
        pltpu.SemaphoreType.DMA,
        pltpu.SemaphoreType.DMA,
        pltpu.SemaphoreType.DMA,
        pltpu.SemaphoreType.DMA,
        pltpu.SemaphoreType.DMA,
        pltpu.SemaphoreType.DMA,
    ],
)(_sc_body)


def kernel(idx, targets, table):
    idxf = idx.reshape(_N)
    tgtf = targets.reshape(_N)
    lse = pl.pallas_call(
        _lse_body,
        out_shape=jax.ShapeDtypeStruct((_VOCAB,), jnp.float32),
    )(table)
    logits2d, part = _sc_gather_loss(table, idxf, tgtf, lse)
    loss = pl.pallas_call(
        _sum_body,
        out_shape=jax.ShapeDtypeStruct((1, 1), jnp.float32),
    )(part)
    return logits2d, loss[0, 0]
